# speedup vs baseline: 1.0085x; 1.0085x over previous
import functools

import jax
import jax.numpy as jnp
from jax.experimental import pallas as pl
from jax.experimental.pallas import tpu as pltpu

B, S, D = 1, 2048, 1024
H, KVH, HD = 16, 4, 64
E, TOPK, F = 8, 2, 768
EPS = 1e-06

BT = 256
NB = S // BT

_dot = functools.partial(jax.lax.dot_general, preferred_element_type=jnp.float32)


def _qkv_body(h_ref, lnw_ref, wqkv_ref, cos_ref, sin_ref, qnw_ref, knw_ref,
              q_ref, k_ref, v_ref):
    h32 = h_ref[...]
    var = jnp.mean(h32 * h32, axis=1, keepdims=True)
    hn = (h32 * jax.lax.rsqrt(var + EPS)) * lnw_ref[...]
    qkv = _dot(hn.astype(jnp.bfloat16), wqkv_ref[...], (((1,), (1,)), ((), ())))
    cos = cos_ref[...]
    sin = sin_ref[...]

    def headnorm_rope(x, w):
        ms = jnp.mean(x * x, axis=1, keepdims=True)
        xn = (x * jax.lax.rsqrt(ms + EPS)) * w
        xr = jnp.concatenate([-xn[:, HD // 2:], xn[:, :HD // 2]], axis=1)
        return xn * cos + xr * sin

    for hh in range(H):
        q = headnorm_rope(qkv[:, hh * HD:(hh + 1) * HD], qnw_ref[...])
        q_ref[hh] = (q * (HD ** -0.5)).astype(jnp.bfloat16)
    for hh in range(KVH):
        k = headnorm_rope(qkv[:, H * HD + hh * HD: H * HD + (hh + 1) * HD],
                          knw_ref[...])
        k_ref[hh] = k.astype(jnp.bfloat16)
    for hh in range(KVH):
        base = (H + KVH) * HD + hh * HD
        v_ref[hh] = qkv[:, base:base + HD].astype(jnp.bfloat16)


def _attn_body(q_ref, k_ref, v_ref, o_ref):
    s = _dot(q_ref[0], k_ref[0], (((1,), (1,)), ((), ())))
    m = jnp.max(s, axis=1, keepdims=True)
    p = jnp.exp(s - m)
    l = jnp.sum(p, axis=1, keepdims=True)
    o = _dot(p.astype(jnp.bfloat16), v_ref[0], (((1,), (0,)), ((), ())))
    o_ref[0] = (o / l).astype(jnp.bfloat16)


def _post_body(attn_ref, wo_ref, res_ref, plw_ref, gate_ref,
               h_ref, x_ref, lg_ref):
    acc = _dot(attn_ref[0], wo_ref[:, 0, :], (((1,), (1,)), ((), ())))
    for hh in range(1, H):
        acc = acc + _dot(attn_ref[hh], wo_ref[:, hh, :], (((1,), (1,)), ((), ())))
    hout = res_ref[...] + acc
    var = jnp.mean(hout * hout, axis=1, keepdims=True)
    xn = (hout * jax.lax.rsqrt(var + EPS)) * plw_ref[...]
    xb = xn.astype(jnp.bfloat16)
    h_ref[...] = hout
    x_ref[...] = xb
    lg_ref[...] = _dot(xb, gate_ref[...], (((1,), (1,)), ((), ())))


def _route_body(lg_ref, wf_ref):
    lg = lg_ref[...]
    m = jnp.max(lg, axis=1, keepdims=True)
    p = jnp.exp(lg - m)
    rw = p / jnp.sum(p, axis=1, keepdims=True)
    lanes = jax.lax.broadcasted_iota(jnp.int32, (S, E), 1)
    m0 = jnp.max(rw, axis=1, keepdims=True)
    i0 = jnp.min(jnp.where(rw == m0, lanes, E), axis=1, keepdims=True)
    sel0 = lanes == i0
    rw2 = jnp.where(sel0, -1.0, rw)
    m1 = jnp.max(rw2, axis=1, keepdims=True)
    i1 = jnp.min(jnp.where(rw2 == m1, lanes, E), axis=1, keepdims=True)
    sel1 = lanes == i1
    wsum = m0 + m1
    wf_ref[...] = (jnp.where(sel0, m0, 0.0) + jnp.where(sel1, m1, 0.0)) / wsum


def _moe_body(x_ref, wg_ref, wu_ref, wd_ref, wf_ref, res_ref, o_ref, acc_ref):
    e = pl.program_id(1)
    x = x_ref[...]
    g = _dot(x, wg_ref[0], (((1,), (1,)), ((), ())))
    u = _dot(x, wu_ref[0], (((1,), (1,)), ((), ())))
    hexp = ((g * jax.nn.sigmoid(g)) * u).astype(jnp.bfloat16)
    o = _dot(hexp, wd_ref[0], (((1,), (1,)), ((), ())))
    lanes = jax.lax.broadcasted_iota(jnp.int32, (BT, E), 1)
    wcol = jnp.sum(jnp.where(lanes == e, wf_ref[...], 0.0), axis=1, keepdims=True)
    contrib = wcol * o

    @pl.when(e == 0)
    def _():
        acc_ref[...] = res_ref[...] + contrib

    @pl.when(e > 0)
    def _():
        acc_ref[...] = acc_ref[...] + contrib

    @pl.when(e == E - 1)
    def _():
        o_ref[...] = acc_ref[...]


def kernel(hidden_states, start_pos, position_cos, position_sin, attention_mask,
           Wq, Wk, Wv, Wo, q_norm_w, k_norm_w, input_ln_w, post_ln_w,
           gate_w, Wg, Wu, Wd):
    x2d = hidden_states.reshape(S, D)
    wqkv = jnp.concatenate([Wq, Wk, Wv], axis=0).astype(jnp.bfloat16)
    wo3 = Wo.reshape(D, H, HD).astype(jnp.bfloat16)
    gate_b = gate_w.astype(jnp.bfloat16)
    wg_b = Wg.astype(jnp.bfloat16)
    wu_b = Wu.astype(jnp.bfloat16)
    wd_b = Wd.astype(jnp.bfloat16)
    lnw = input_ln_w.reshape(1, D)
    plw = post_ln_w.reshape(1, D)
    qnw = q_norm_w.reshape(1, HD)
    knw = k_norm_w.reshape(1, HD)

    q3, k3, v3 = pl.pallas_call(
        _qkv_body,
        grid=(NB,),
        in_specs=[
            pl.BlockSpec((BT, D), lambda i: (i, 0)),
            pl.BlockSpec((1, D), lambda i: (0, 0)),
            pl.BlockSpec(((H + 2 * KVH) * HD, D), lambda i: (0, 0)),
            pl.BlockSpec((BT, HD), lambda i: (i, 0)),
            pl.BlockSpec((BT, HD), lambda i: (i, 0)),
            pl.BlockSpec((1, HD), lambda i: (0, 0)),
            pl.BlockSpec((1, HD), lambda i: (0, 0)),
        ],
        out_specs=[
            pl.BlockSpec((H, BT, HD), lambda i: (0, i, 0)),
            pl.BlockSpec((KVH, BT, HD), lambda i: (0, i, 0)),
            pl.BlockSpec((KVH, BT, HD), lambda i: (0, i, 0)),
        ],
        out_shape=[
            jax.ShapeDtypeStruct((H, S, HD), jnp.bfloat16),
            jax.ShapeDtypeStruct((KVH, S, HD), jnp.bfloat16),
            jax.ShapeDtypeStruct((KVH, S, HD), jnp.bfloat16),
        ],
    )(x2d, lnw, wqkv, position_cos, position_sin, qnw, knw)

    attn3 = pl.pallas_call(
        _attn_body,
        grid=(H, NB),
        in_specs=[
            pl.BlockSpec((1, BT, HD), lambda h, i: (h, i, 0)),
            pl.BlockSpec((1, S, HD), lambda h, i: (h // (H // KVH), 0, 0)),
            pl.BlockSpec((1, S, HD), lambda h, i: (h // (H // KVH), 0, 0)),
        ],
        out_specs=pl.BlockSpec((1, BT, HD), lambda h, i: (h, i, 0)),
        out_shape=jax.ShapeDtypeStruct((H, S, HD), jnp.bfloat16),
    )(q3, k3, v3)

    hres, xb, logits = pl.pallas_call(
        _post_body,
        grid=(NB,),
        in_specs=[
            pl.BlockSpec((H, BT, HD), lambda i: (0, i, 0)),
            pl.BlockSpec((D, H, HD), lambda i: (0, 0, 0)),
            pl.BlockSpec((BT, D), lambda i: (i, 0)),
            pl.BlockSpec((1, D), lambda i: (0, 0)),
            pl.BlockSpec((E, D), lambda i: (0, 0)),
        ],
        out_specs=[
            pl.BlockSpec((BT, D), lambda i: (i, 0)),
            pl.BlockSpec((BT, D), lambda i: (i, 0)),
            pl.BlockSpec((BT, E), lambda i: (i, 0)),
        ],
        out_shape=[
            jax.ShapeDtypeStruct((S, D), jnp.float32),
            jax.ShapeDtypeStruct((S, D), jnp.bfloat16),
            jax.ShapeDtypeStruct((S, E), jnp.float32),
        ],
    )(attn3, wo3, x2d, plw, gate_b)

    wfull = pl.pallas_call(
        _route_body,
        grid=(1,),
        in_specs=[pl.BlockSpec((S, E), lambda i: (0, 0))],
        out_specs=pl.BlockSpec((S, E), lambda i: (0, 0)),
        out_shape=jax.ShapeDtypeStruct((S, E), jnp.float32),
    )(logits)

    out = pl.pallas_call(
        _moe_body,
        grid=(NB, E),
        in_specs=[
            pl.BlockSpec((BT, D), lambda i, e: (i, 0)),
            pl.BlockSpec((1, F, D), lambda i, e: (e, 0, 0)),
            pl.BlockSpec((1, F, D), lambda i, e: (e, 0, 0)),
            pl.BlockSpec((1, D, F), lambda i, e: (e, 0, 0)),
            pl.BlockSpec((BT, E), lambda i, e: (i, 0)),
            pl.BlockSpec((BT, D), lambda i, e: (i, 0)),
        ],
        out_specs=pl.BlockSpec((BT, D), lambda i, e: (i, 0)),
        out_shape=jax.ShapeDtypeStruct((S, D), jnp.float32),
        scratch_shapes=[pltpu.VMEM((BT, D), jnp.float32)],
    )(xb, wg_b, wu_b, wd_b, wfull, hres)

    return out.reshape(B, S, D)
